# Initial kernel scaffold; baseline (speedup 1.0000x reference)
import jax
import jax.numpy as jnp
from jax.experimental import pallas as pl


def kernel(adj, gc_W, gc_b, W1, b1, W2, b2, W3, b3):
    raise NotImplementedError("write your pallas kernel here")


# speedup vs baseline: 1.0034x; 1.0034x over previous
import jax
import jax.numpy as jnp
from jax.experimental import pallas as pl
from jax.experimental.pallas import tpu as pltpu

N = 2048
D_SIZE = 128
GC_SIZE = 256
HID = GC_SIZE // 2
BLK = 256
NBLK = N // BLK


def _fused_kernel(adj_ref, x_ref, gcW_ref, gcb_ref, W1_ref, b1_ref,
                  W2_ref, b2_ref, W3_ref, b3_ref, out_ref, S_scr, acc_scr):
    i = pl.program_id(0)

    @pl.when(i == 0)
    def _init():
        S_scr[...] = jax.lax.dot(
            x_ref[...], gcW_ref[...],
            preferred_element_type=jnp.float32,
            precision=jax.lax.Precision.HIGHEST)
        acc_scr[...] = jnp.zeros_like(acc_scr)

    a = adj_ref[...]
    rowsum = jnp.sum(a, axis=1, keepdims=True)
    rinv = jnp.where(rowsum > 0, 1.0 / rowsum, 0.0)
    y = jax.lax.dot(a, S_scr[...],
                    preferred_element_type=jnp.float32,
                    precision=jax.lax.Precision.HIGHEST)
    h = jnp.maximum(y * rinv + gcb_ref[...], 0.0)
    h = jax.lax.dot(h, W1_ref[...],
                    preferred_element_type=jnp.float32,
                    precision=jax.lax.Precision.HIGHEST) + b1_ref[...]
    h = jnp.where(h >= 0, h, 0.2 * h)
    h = jax.lax.dot(h, W2_ref[...],
                    preferred_element_type=jnp.float32,
                    precision=jax.lax.Precision.HIGHEST) + b2_ref[...]
    h = jnp.where(h >= 0, h, 0.2 * h)
    acc_scr[...] += jnp.sum(h, axis=0, keepdims=True)

    @pl.when(i == NBLK - 1)
    def _fin():
        s = jax.lax.dot(acc_scr[...], W3_ref[...],
                        preferred_element_type=jnp.float32,
                        precision=jax.lax.Precision.HIGHEST) + b3_ref[...]
        out_ref[...] = jax.nn.sigmoid(s)


def kernel(adj, gc_W, gc_b, W1, b1, W2, b2, W3, b3):
    A = 0.5 * (adj + adj.T)
    deg = jnp.sum(A, axis=1)
    L = jnp.diag(deg) - A
    _, v = jnp.linalg.eigh(L)
    x = v[:, :D_SIZE]

    out = pl.pallas_call(
        _fused_kernel,
        grid=(NBLK,),
        in_specs=[
            pl.BlockSpec((BLK, N), lambda i: (i, 0)),
            pl.BlockSpec((N, D_SIZE), lambda i: (0, 0)),
            pl.BlockSpec((D_SIZE, GC_SIZE), lambda i: (0, 0)),
            pl.BlockSpec((1, GC_SIZE), lambda i: (0, 0)),
            pl.BlockSpec((GC_SIZE, HID), lambda i: (0, 0)),
            pl.BlockSpec((1, HID), lambda i: (0, 0)),
            pl.BlockSpec((HID, 8), lambda i: (0, 0)),
            pl.BlockSpec((1, 8), lambda i: (0, 0)),
            pl.BlockSpec((8, 1), lambda i: (0, 0)),
            pl.BlockSpec((1, 1), lambda i: (0, 0)),
        ],
        out_specs=pl.BlockSpec((1, 1), lambda i: (0, 0)),
        out_shape=jax.ShapeDtypeStruct((1, 1), jnp.float32),
        scratch_shapes=[
            pltpu.VMEM((N, GC_SIZE), jnp.float32),
            pltpu.VMEM((1, 8), jnp.float32),
        ],
    )(adj, x, gc_W, gc_b.reshape(1, GC_SIZE), W1, b1.reshape(1, HID),
      W2, b2.reshape(1, 8), W3, b3.reshape(1, 1))
    return out.reshape(1)


# speedup vs baseline: 1.0036x; 1.0003x over previous
import jax
import jax.numpy as jnp
from jax.experimental import pallas as pl
from jax.experimental.pallas import tpu as pltpu

N = 2048
D_SIZE = 128
GC_SIZE = 256
HID = GC_SIZE // 2
BLK = 256
NBLK = N // BLK


def _fused_kernel(adj_ref, rinv_ref, x_ref, gcW_ref, gcb_ref, W1_ref,
                  b1_ref, W2_ref, b2_ref, out_ref, S_scr):
    i = pl.program_id(0)

    @pl.when(i == 0)
    def _init():
        S_scr[...] = jax.lax.dot(
            x_ref[...], gcW_ref[...],
            preferred_element_type=jnp.float32)

    a_n = adj_ref[...] * rinv_ref[...]
    y = jax.lax.dot(a_n, S_scr[...],
                    preferred_element_type=jnp.float32)
    h = jnp.maximum(y + gcb_ref[...], 0.0)
    h = jax.lax.dot(h, W1_ref[...],
                    preferred_element_type=jnp.float32) + b1_ref[...]
    h = jnp.where(h >= 0, h, 0.2 * h)
    h = jax.lax.dot(h, W2_ref[...],
                    preferred_element_type=jnp.float32) + b2_ref[...]
    out_ref[...] = jnp.where(h >= 0, h, 0.2 * h)


def kernel(adj, gc_W, gc_b, W1, b1, W2, b2, W3, b3):
    A = 0.5 * (adj + adj.T)
    deg = jnp.sum(A, axis=1)
    L = jnp.diag(deg) - A
    _, v = jnp.linalg.eigh(L)
    x = v[:, :D_SIZE]

    rowsum = jnp.sum(adj, axis=1)
    r_inv = jnp.where(rowsum > 0, 1.0 / rowsum, 0.0)

    h2 = pl.pallas_call(
        _fused_kernel,
        grid=(NBLK,),
        in_specs=[
            pl.BlockSpec((BLK, N), lambda i: (i, 0)),
            pl.BlockSpec((BLK, 1), lambda i: (i, 0)),
            pl.BlockSpec((N, D_SIZE), lambda i: (0, 0)),
            pl.BlockSpec((D_SIZE, GC_SIZE), lambda i: (0, 0)),
            pl.BlockSpec((1, GC_SIZE), lambda i: (0, 0)),
            pl.BlockSpec((GC_SIZE, HID), lambda i: (0, 0)),
            pl.BlockSpec((1, HID), lambda i: (0, 0)),
            pl.BlockSpec((HID, 8), lambda i: (0, 0)),
            pl.BlockSpec((1, 8), lambda i: (0, 0)),
        ],
        out_specs=pl.BlockSpec((BLK, 8), lambda i: (i, 0)),
        out_shape=jax.ShapeDtypeStruct((N, 8), jnp.float32),
        scratch_shapes=[
            pltpu.VMEM((N, GC_SIZE), jnp.float32),
        ],
    )(adj, r_inv.reshape(N, 1), x, gc_W, gc_b.reshape(1, GC_SIZE),
      W1, b1.reshape(1, HID), W2, b2.reshape(1, 8))

    s = jnp.sum(h2, axis=0)
    return jax.nn.sigmoid(s @ W3 + b3)
